# baseline (speedup 1.0000x reference)
import dataclasses
import functools

import jax
import jax.numpy as jnp
from jax import lax
from jax.experimental import pallas as pl
from jax.experimental.pallas import tpu as pltpu
from jax.experimental.pallas import tpu_sc as plsc

_B, _NN, _LB, _TRAJ = 64, 10000, 8, 64
_NP = 10240


def _tc_a(x, fc1_W, fc1_b):
    def body(fb_ref, x_ref, fw_ref, o_ref):
        w = fw_ref[0, :]
        fb = fb_ref[0]
        for r in range(8):
            row = jnp.sum(x_ref[r] * w[:, None], axis=0) + fb
            o_ref[pl.ds(r * _NP, _NN)] = row

    return pl.pallas_call(
        body,
        grid=(8,),
        in_specs=[
            pl.BlockSpec(memory_space=pltpu.SMEM),
            pl.BlockSpec((8, _LB, _NN), lambda i: (i, 0, 0)),
            pl.BlockSpec((1, _LB), lambda i: (0, 0)),
        ],
        out_specs=pl.BlockSpec((8 * _NP,), lambda i: (i,)),
        out_shape=jax.ShapeDtypeStruct((_B * _NP,), jnp.float32),
    )(fc1_b, x, fc1_W)


def _sc_kernel(stops, out_flat, table, pack, x_week, x_feat_flat,
               emb_week_flat):
    mesh = plsc.VectorSubcoreMesh(core_axis_name="c", subcore_axis_name="s")
    cp = pltpu.CompilerParams(use_tc_tiling_on_sc=False)
    if "needs_layout_passes" in pltpu.CompilerParams.__dataclass_fields__:
        cp = dataclasses.replace(cp, needs_layout_passes=False)

    @functools.partial(
        pl.kernel,
        mesh=mesh,
        out_type=jax.ShapeDtypeStruct((_B * _NP,), jnp.float32),
        scratch_types=[
            pltpu.VMEM((_TRAJ,), jnp.int32),
            pltpu.VMEM((_TRAJ,), jnp.int32),
            pltpu.VMEM((_TRAJ, 16), jnp.float32),
            pltpu.VMEM((_TRAJ,), jnp.float32),
            pltpu.VMEM((_NN,), jnp.float32),
            pltpu.VMEM((_NN,), jnp.float32),
            pltpu.VMEM((_NN,), jnp.float32),
            pltpu.VMEM((_NN,), jnp.float32),
            pltpu.VMEM((144,), jnp.float32),
            pltpu.VMEM((64,), jnp.int32),
            pltpu.VMEM((128,), jnp.float32),
            pltpu.VMEM((240,), jnp.float32),
            pltpu.SemaphoreType.DMA,
            pltpu.SemaphoreType.DMA,
            pltpu.SemaphoreType.DMA,
        ],
        compiler_params=cp,
    )
    def sck(stops_hbm, oflat_hbm, table_hbm, pack_hbm, xw_hbm, xf_hbm,
            ew_hbm, sage_hbm,
            sb, oidx, rows, osrc_v, aggl0, cntl0, aggl1, cntl1, packb,
            xwbuf, xfbuf, ewbuf, sem_rows, sem_osrc, sem_out):
        wid = lax.axis_index("s") * 2 + lax.axis_index("c")
        pltpu.sync_copy(pack_hbm, packb)
        pltpu.sync_copy(xw_hbm, xwbuf)
        pltpu.sync_copy(xf_hbm, xfbuf)
        pltpu.sync_copy(ew_hbm, ewbuf)
        iota = lax.iota(jnp.int32, 16)
        p0 = packb[pl.ds(0, 16)]
        wl48 = jnp.sum(jnp.where(iota == 0, p0, 0.0))
        wl49 = jnp.sum(jnp.where(iota == 1, p0, 0.0))
        blv = jnp.sum(jnp.where(iota == 2, p0, 0.0))
        ones16 = jnp.ones((16,), jnp.float32)
        lane12 = jnp.full((16,), 12, jnp.int32)
        masks = [(iota == j) for j in range(16)]
        wlA = packb[pl.ds(16, 16)]
        wlB = packb[pl.ds(32, 16)]
        wlC = packb[pl.ds(48, 16)]
        wlE = packb[pl.ds(64, 16)]
        wrA = packb[pl.ds(80, 16)]
        wrB = packb[pl.ds(96, 16)]
        wrC = packb[pl.ds(112, 16)]
        wrE = packb[pl.ds(128, 16)]

        out_copies = []
        for r, (aggl, cntl) in enumerate(((aggl0, cntl0), (aggl1, cntl1))):
            b = wid * 2 + r
            bv = jnp.full((16,), 0, jnp.int32) + b
            pltpu.sync_copy(stops_hbm.at[b], sb)
            rows_cp = pltpu.async_copy(table_hbm.at[sb], rows, sem_rows)
            boff = b * _NP
            for u in range(4):
                oidx[pl.ds(u * 16, 16)] = sb[pl.ds(u * 16, 16)] + boff
            osrc_cp = pltpu.async_copy(oflat_hbm.at[oidx], osrc_v, sem_osrc)
            rows_cp.wait()
            acc = jnp.zeros((16,), jnp.float32)
            for t in range(_TRAJ):
                acc = acc + rows[t]

            wv = plsc.load_gather(xwbuf, [bv]) * 34
            ew0 = plsc.load_gather(ewbuf, [wv + iota])
            ew1 = plsc.load_gather(ewbuf, [wv + (iota + 16)])
            ew2 = plsc.load_gather(
                ewbuf, [jnp.minimum(wv + (iota + 32), 237)])
            xfg = plsc.load_gather(
                xfbuf, [jnp.clip(iota + (2 * b - 2), 0, 127)])
            chunk2 = jnp.where(iota < 2, ew2,
                               jnp.where(iota < 4, xfg, 0.0))
            cl = jnp.sum(ew0 * wlA + ew1 * wlB + chunk2 * wlC + acc * wlE)
            crbl = blv + jnp.sum(
                ew0 * wrA + ew1 * wrB + chunk2 * wrC + acc * wrE)

            @pl.loop(0, _NN, step=400)
            def _(i):
                z = jnp.zeros((16,), jnp.float32)
                for u in range(25):
                    aggl[pl.ds(i + u * 16, 16)] = z
                    cntl[pl.ds(i + u * 16, 16)] = z

            osrc_cp.wait()
            for c in range(4):
                toff = c * 16
                osrc = osrc_v[pl.ds(toff, 16)]
                xdv = plsc.load_gather(rows, [iota + toff, lane12])
                v = osrc * wl48 + xdv * wl49
                valid = (iota + toff) < (_TRAJ - 1)
                dstv = plsc.load_gather(
                    sb, [jnp.minimum(iota + (toff + 1), _TRAJ - 1)])
                for j in range(16):
                    m = valid & masks[j]
                    plsc.addupdate_scatter(aggl, [dstv], v, mask=m)
                    plsc.addupdate_scatter(cntl, [dstv], ones16, mask=m)

            @pl.loop(0, _NN, step=80)
            def _(i):
                for u in range(5):
                    sl = pl.ds(i + u * 16, 16)
                    cn = cntl[sl]
                    ag = aggl[sl]
                    mean = ag / jnp.maximum(cn, 1.0)
                    aggl[sl] = (mean + jnp.where(cn >= 0.5, cl, 0.0)
                                + crbl)

            out_copies.append(pltpu.async_copy(
                aggl, sage_hbm.at[pl.ds(boff, _NN)], sem_out))

        for c in out_copies:
            c.wait()

    return sck(stops, out_flat, table, pack, x_week, x_feat_flat,
               emb_week_flat)


def _tc_b(out_flat, sage_flat, x_mask, x_dist2, Wr):
    def body(out_ref, sage_ref, mask_ref, xd_ref, wr_ref, o_ref):
        wr = wr_ref[0, :]
        wr48 = wr[48:49]
        wr49 = wr[49:50]
        xd = xd_ref[0, :]
        for r in range(8):
            sl = pl.ds(r * _NP, _NN)
            g = (sage_ref[sl] + out_ref[sl] * wr48 + xd * wr49)
            gm = jnp.max(g)
            e = jnp.exp(g - gm)
            logp = (g - gm) - jnp.log(jnp.sum(e))
            o_ref[r, :] = jnp.where(mask_ref[r, :] != 0, -1e8, logp)

    return pl.pallas_call(
        body,
        grid=(8,),
        in_specs=[
            pl.BlockSpec((8 * _NP,), lambda i: (i,)),
            pl.BlockSpec((8 * _NP,), lambda i: (i,)),
            pl.BlockSpec((8, _NN), lambda i: (i, 0)),
            pl.BlockSpec((1, _NN), lambda i: (0, 0)),
            pl.BlockSpec((1, 50), lambda i: (0, 0)),
        ],
        out_specs=pl.BlockSpec((8, _NN), lambda i: (i, 0)),
        out_shape=jax.ShapeDtypeStruct((_B, _NN), jnp.float32),
    )(out_flat, sage_flat, x_mask, x_dist2, Wr)


def kernel(stops, x, x_dist, x_features, x_week, x_mask, emb_week, emb_stop,
           fc1_W, fc1_b, Wl, bl, Wr):
    f32 = jnp.float32
    stops32 = stops.astype(jnp.int32)
    x_week32 = x_week.astype(jnp.int32)
    table = jnp.concatenate(
        [emb_stop, x_dist[:, None], jnp.zeros((_NN, 3), f32)], axis=1)
    z12 = jnp.zeros((12,), f32)
    z4 = jnp.zeros((4,), f32)
    pack = jnp.concatenate([
        Wl[0, 48:50], bl, jnp.zeros((13,), f32),
        Wl[0, 0:16], Wl[0, 16:32], Wl[0, 32:36], z12, Wl[0, 36:48], z4,
        Wr[0, 0:16], Wr[0, 16:32], Wr[0, 32:36], z12, Wr[0, 36:48], z4,
    ])
    xf_flat = x_features.reshape(-1).astype(f32)
    ew_flat = jnp.concatenate([emb_week.reshape(-1), jnp.zeros((2,), f32)])

    out_flat = _tc_a(x, fc1_W, fc1_b)
    sage_flat = _sc_kernel(stops32, out_flat, table, pack, x_week32,
                           xf_flat, ew_flat)
    return _tc_b(out_flat, sage_flat, x_mask, x_dist.reshape(1, _NN), Wr)
